# baseline (speedup 1.0000x reference)
import jax
import jax.numpy as jnp
from jax.experimental import pallas as pl
from jax.experimental.pallas import tpu as pltpu

_FILTERS = 64
_C = 64
_D = _H = _W = 32
_PZ, _PY, _PX = 34, 34, 48
_PLANE = _PY * _PX
_NPAD = _PZ * _PLANE
_NROW = 56640
_R0 = 1680
_TB = 2048
_G = 26
_L = _TB * _G

_TAPS = tuple(((kk // 9 - 1) * _PLANE + ((kk // 3) % 3 - 1) * _PX, kk % 3 - 1)
              for kk in range(27))


def _conv_body(fstk_ref, w_ref, b_ref, mask_ref, out_ref, fscr_ref, sem):
    g = pl.program_id(0)

    @pl.when(g == 0)
    def _load():
        cp = pltpu.make_async_copy(fstk_ref, fscr_ref, sem)
        cp.start()
        cp.wait()

    base = _R0 + g * _TB
    acc = None
    for kk, (off, dx) in enumerate(_TAPS):
        part = jnp.dot(fscr_ref[dx + 1, pl.ds(base + off, _TB), :], w_ref[kk],
                       preferred_element_type=jnp.float32)
        acc = part if acc is None else acc + part
    out_ref[...] = (acc + b_ref[...]) * mask_ref[...].astype(jnp.float32)


def kernel(feat, index, w, b):
    f = feat.reshape(_D, _H, _W, _C).astype(jnp.bfloat16)
    fp3 = jnp.pad(f, ((1, 1), (1, 1), (1, 15), (0, 0))).reshape(_NPAD, _C)
    fp3 = jnp.pad(fp3, ((0, _NROW - _NPAD), (0, 0)))
    fm = jnp.pad(fp3[:-1], ((1, 0), (0, 0)))
    fpp = jnp.pad(fp3[1:], ((0, 1), (0, 0)))
    fstk = jnp.stack([fm, fp3, fpp])
    m = (index.reshape(_D, _H, _W) != 0).astype(jnp.bfloat16)
    mp = jnp.pad(m, ((1, 1), (1, 1), (1, 15))).reshape(_NPAD)
    maskb = jnp.broadcast_to(mp[_R0:_R0 + _L, None], (_L, _FILTERS))
    w27 = w.reshape(27, _C, _FILTERS).astype(jnp.bfloat16)

    out = pl.pallas_call(
        _conv_body,
        grid=(_G,),
        in_specs=[
            pl.BlockSpec(memory_space=pltpu.MemorySpace.HBM),
            pl.BlockSpec((27, _C, _FILTERS), lambda g: (0, 0, 0)),
            pl.BlockSpec((1, _FILTERS), lambda g: (0, 0)),
            pl.BlockSpec((_TB, _FILTERS), lambda g: (g, 0)),
        ],
        out_specs=pl.BlockSpec((_TB, _FILTERS), lambda g: (g, 0)),
        out_shape=jax.ShapeDtypeStruct((_L, _FILTERS), jnp.float32),
        scratch_shapes=[
            pltpu.VMEM((3, _NROW, _C), jnp.bfloat16),
            pltpu.SemaphoreType.DMA,
        ],
    )(fstk, w27, b.reshape(1, _FILTERS), maskb)

    full = jnp.pad(out, ((_R0, _NPAD - _R0 - _L), (0, 0)))
    full = full.reshape(_PZ, _PY, _PX, _FILTERS)[1:33, 1:33, 1:33]
    return full.reshape(1, _D, _H, _W, _FILTERS)
